# baseline (speedup 1.0000x reference)
import functools

import jax
import jax.numpy as jnp
from jax import lax
from jax.experimental import pallas as pl
from jax.experimental.pallas import tpu as pltpu
from jax.experimental.pallas import tpu_sc as plsc

_NSUB = 16


def _row_split(n):
    ra = ((n // _NSUB) + 7) & ~7
    rb = n - (_NSUB - 1) * ra
    assert rb > 0 and rb % 8 == 0
    return ra, rb


def _zero_rows(zb, acc, sid, n, zr):
    ra, rb = _row_split(n)

    def mk(base, cnt):
        def zcopy(k, carry):
            pltpu.sync_copy(zb, acc.at[pl.ds(base + k * zr, zr)])
            return carry
        lax.fori_loop(0, cnt // zr, zcopy, 0)

    @pl.when(sid < _NSUB - 1)
    def _():
        mk(sid * ra, ra)

    @pl.when(sid == _NSUB - 1)
    def _():
        mk((_NSUB - 1) * ra, rb)


def _drain_rows(acc, p, sid, n):
    ra, rb = _row_split(n)

    @pl.when(sid < _NSUB - 1)
    def _():
        pltpu.sync_copy(acc.at[pl.ds(sid * ra, ra)], p.at[pl.ds(sid * ra, ra)])

    @pl.when(sid == _NSUB - 1)
    def _():
        pltpu.sync_copy(acc.at[pl.ds((_NSUB - 1) * ra, rb)],
                        p.at[pl.ds((_NSUB - 1) * ra, rb)])


def _lane_bcast(w16, lane):
    return w16.at[jnp.full((16,), lane, jnp.int32)].get(
        mode="promise_in_bounds")


def _groups(B):
    out, s = [], 0
    while s + 16 <= B:
        out.append((s, 0))
        s += 16
    if s < B:
        out.append((B - 16, 16 - (B - s)))
    return out


def _zero_zb(zb, zr):
    zero16 = jnp.zeros((16,), jnp.float32)
    for r in range(zr):
        for j in range(8):
            zb.at[r][pl.ds(j * 16, 16)] = zero16


_NS = 3


def _edge_pipeline(y, rows_v, col_hbm, w_hbm, acc, colbs, wbs, gbs,
                   sgs, sws, scs, sss, e0, ept, B, gather):
    nb = ept // B
    assert nb >= _NS + 1

    def issue(t, s):
        pltpu.async_copy(col_hbm.at[pl.ds(e0 + t * B, B)], colbs[s], scs[s])
        pltpu.async_copy(w_hbm.at[pl.ds(e0 + t * B, B)], wbs[s], sws[s])
        if gather:
            pltpu.async_copy(y.at[rows_v.at[pl.ds(t * B, B)]], gbs[s], sgs[s])

    def wait_gather(s):
        pltpu.make_async_copy(y.at[pl.ds(0, B)], gbs[s], sgs[s]).wait()

    def wait_col(s):
        pltpu.make_async_copy(col_hbm.at[pl.ds(0, B)], colbs[s], scs[s]).wait()

    def wait_w(s):
        pltpu.make_async_copy(w_hbm.at[pl.ds(0, B)], wbs[s], sws[s]).wait()

    def wait_scat(s):
        pltpu.make_async_copy(gbs[s], acc.at[pl.ds(0, B)], sss[s]).wait()

    def batch(u, s, in_loop):
        if gather:
            wait_gather(s)
        v = u + 2
        s2 = (s + 2) % _NS
        if in_loop:
            @pl.when(v < nb)
            def _():
                @pl.when(v >= _NS)
                def _():
                    wait_scat(s2)
                issue(v, s2)
        else:
            if v < nb:
                if v >= _NS:
                    wait_scat(s2)
                issue(v, s2)
        wait_w(s)
        for start, lo in _groups(B):
            w16 = wbs[s][pl.ds(start, 16)]
            for lane in range(lo, 16):
                wv = _lane_bcast(w16, lane)
                r = gbs[s].at[start + lane]
                if gather:
                    for j in range(8):
                        r[pl.ds(j * 16, 16)] = r[pl.ds(j * 16, 16)] * wv
                else:
                    r[pl.ds(0, 16)] = wv
        wait_col(s)
        pltpu.async_copy(gbs[s], acc.at[colbs[s]], sss[s], add=True)

    for t in range(2):
        issue(t, t)

    def tri(q, carry):
        for s in range(_NS):
            batch(q * _NS + s, s, True)
        return carry
    nq = nb // _NS
    lax.fori_loop(0, nq, tri, 0)
    for k in range(nb % _NS):
        batch(nq * _NS + k, k, False)
    for s in range(_NS):
        wait_scat(s)


@functools.lru_cache(maxsize=None)
def _sc_deg(n, e):
    ncores = 2
    ept = e // (_NSUB * ncores)
    B = 40
    nb = ept // B
    zr = 8
    mesh = plsc.VectorSubcoreMesh(core_axis_name="c", subcore_axis_name="s")

    def body(col_hbm, w_hbm, p0, p1, *scr):
        colbs, wbs, gbs = scr[0:3], scr[3:6], scr[6:9]
        zb, acc = scr[9], scr[10]
        sws, scs, sss = scr[11:14], scr[14:17], scr[17:20]
        cid = lax.axis_index("c")
        sid = lax.axis_index("s")
        _zero_zb(zb, zr)
        _zero_rows(zb, acc, sid, n, zr)
        plsc.subcore_barrier()

        wid = cid * _NSUB + sid
        e0 = wid * ept
        _edge_pipeline(None, None, col_hbm, w_hbm, acc,
                       colbs, wbs, gbs,
                       None, sws, scs, sss, e0, ept, B, gather=False)
        plsc.subcore_barrier()

        @pl.when(cid == 0)
        def _():
            _drain_rows(acc, p0, sid, n)

        @pl.when(cid == 1)
        def _():
            _drain_rows(acc, p1, sid, n)

    return pl.kernel(
        body,
        mesh=mesh,
        out_type=[jax.ShapeDtypeStruct((n, 128), jnp.float32)] * 2,
        scratch_types=(
            [pltpu.VMEM((B,), jnp.int32)] * 3
            + [pltpu.VMEM((B,), jnp.float32)] * 3
            + [pltpu.VMEM((B, 128), jnp.float32)] * 3
            + [pltpu.VMEM((zr, 128), jnp.float32),
               pltpu.VMEM_SHARED((n, 128), jnp.float32)]
            + [pltpu.SemaphoreType.DMA] * 9
        ),
    )


@functools.lru_cache(maxsize=None)
def _sc_agg_csplit(n, e):
    ept = e // _NSUB
    B = 80
    zr = 8
    mesh = plsc.VectorSubcoreMesh(core_axis_name="c", subcore_axis_name="s")

    def body(y0, y1, row_hbm, col_hbm, w_hbm, p0, p1, *scr):
        rows_v = scr[0]
        colbs, wbs, gbs = scr[1:4], scr[4:7], scr[7:10]
        zb, acc = scr[10], scr[11]
        sgs, sws, scs, sss = (scr[12:15], scr[15:18], scr[18:21],
                              scr[21:24])
        cid = lax.axis_index("c")
        sid = lax.axis_index("s")
        _zero_zb(zb, zr)
        _zero_rows(zb, acc, sid, n, zr)

        e0 = sid * ept
        pltpu.sync_copy(row_hbm.at[pl.ds(e0, ept)], rows_v)
        plsc.subcore_barrier()

        def run(y):
            _edge_pipeline(y, rows_v, col_hbm, w_hbm, acc,
                           colbs, wbs, gbs,
                           sgs, sws, scs, sss, e0, ept, B, gather=True)

        @pl.when(cid == 0)
        def _():
            run(y0)

        @pl.when(cid == 1)
        def _():
            run(y1)
        plsc.subcore_barrier()

        @pl.when(cid == 0)
        def _():
            _drain_rows(acc, p0, sid, n)

        @pl.when(cid == 1)
        def _():
            _drain_rows(acc, p1, sid, n)

    return pl.kernel(
        body,
        mesh=mesh,
        out_type=[jax.ShapeDtypeStruct((n, 128), jnp.float32)] * 2,
        scratch_types=(
            [pltpu.VMEM((ept,), jnp.int32)]
            + [pltpu.VMEM((B,), jnp.int32)] * 3
            + [pltpu.VMEM((B,), jnp.float32)] * 3
            + [pltpu.VMEM((B, 128), jnp.float32)] * 3
            + [pltpu.VMEM((zr, 128), jnp.float32),
               pltpu.VMEM_SHARED((n, 128), jnp.float32)]
            + [pltpu.SemaphoreType.DMA] * 12
        ),
    )


@functools.lru_cache(maxsize=None)
def _sc_agg_esplit(n, e):
    ncores = 2
    ept = e // (_NSUB * ncores)
    B = 40
    zr = 8
    mesh = plsc.VectorSubcoreMesh(core_axis_name="c", subcore_axis_name="s")

    def body(y, row_hbm, col_hbm, w_hbm, p0, p1, *scr):
        rows_v = scr[0]
        colbs, wbs, gbs = scr[1:4], scr[4:7], scr[7:10]
        zb, acc = scr[10], scr[11]
        sgs, sws, scs, sss = (scr[12:15], scr[15:18], scr[18:21],
                              scr[21:24])
        cid = lax.axis_index("c")
        sid = lax.axis_index("s")
        _zero_zb(zb, zr)
        _zero_rows(zb, acc, sid, n, zr)

        wid = cid * _NSUB + sid
        e0 = wid * ept
        pltpu.sync_copy(row_hbm.at[pl.ds(e0, ept)], rows_v)
        plsc.subcore_barrier()

        _edge_pipeline(y, rows_v, col_hbm, w_hbm, acc,
                       colbs, wbs, gbs,
                       sgs, sws, scs, sss, e0, ept, B, gather=True)
        plsc.subcore_barrier()

        @pl.when(cid == 0)
        def _():
            _drain_rows(acc, p0, sid, n)

        @pl.when(cid == 1)
        def _():
            _drain_rows(acc, p1, sid, n)

    return pl.kernel(
        body,
        mesh=mesh,
        out_type=[jax.ShapeDtypeStruct((n, 128), jnp.float32)] * 2,
        scratch_types=(
            [pltpu.VMEM((ept,), jnp.int32)]
            + [pltpu.VMEM((B,), jnp.int32)] * 3
            + [pltpu.VMEM((B,), jnp.float32)] * 3
            + [pltpu.VMEM((B, 128), jnp.float32)] * 3
            + [pltpu.VMEM((zr, 128), jnp.float32),
               pltpu.VMEM_SHARED((n, 128), jnp.float32)]
            + [pltpu.SemaphoreType.DMA] * 12
        ),
    )


def _dis(d0, d1):
    return lax.rsqrt(d0[:, 0:1] + d1[:, 0:1] + 1.0)


@functools.lru_cache(maxsize=None)
def _tc_pre(n, in_ch, bm):
    half = in_ch // 2

    def body(x_ref, d0, d1, xs0, xs1):
        xs = x_ref[...] * _dis(d0, d1)
        xs0[...] = xs[:, :half]
        xs1[...] = xs[:, half:]

    return pl.pallas_call(
        body,
        grid=(n // bm,),
        in_specs=[
            pl.BlockSpec((bm, in_ch), lambda m: (m, 0)),
            pl.BlockSpec((bm, 128), lambda m: (m, 0)),
            pl.BlockSpec((bm, 128), lambda m: (m, 0)),
        ],
        out_specs=[pl.BlockSpec((bm, half), lambda m: (m, 0))] * 2,
        out_shape=[jax.ShapeDtypeStruct((n, half), jnp.float32)] * 2,
    )


@functools.lru_cache(maxsize=None)
def _tc_l12(n, c, h1, h2, bm):
    def body(p0, p1, x0, x1, d0, d1, w1, bb1, w2, o0, o1):
        dis = _dis(d0, d1)
        a = jnp.concatenate([p0[...] + x0[...], p1[...] + x1[...]], axis=1) * dis
        hh = jnp.dot(a, w1[...], preferred_element_type=jnp.float32) + bb1[...]
        g = jnp.dot(hh, w2[...], preferred_element_type=jnp.float32) * dis
        o0[...] = g[:, : h2 // 2]
        o1[...] = g[:, h2 // 2:]

    return pl.pallas_call(
        body,
        grid=(n // bm,),
        in_specs=[
            pl.BlockSpec((bm, c), lambda m: (m, 0)),
            pl.BlockSpec((bm, c), lambda m: (m, 0)),
            pl.BlockSpec((bm, c), lambda m: (m, 0)),
            pl.BlockSpec((bm, c), lambda m: (m, 0)),
            pl.BlockSpec((bm, 128), lambda m: (m, 0)),
            pl.BlockSpec((bm, 128), lambda m: (m, 0)),
            pl.BlockSpec((2 * c, h1), lambda m: (0, 0)),
            pl.BlockSpec((1, h1), lambda m: (0, 0)),
            pl.BlockSpec((h1, h2), lambda m: (0, 0)),
        ],
        out_specs=[pl.BlockSpec((bm, h2 // 2), lambda m: (m, 0))] * 2,
        out_shape=[jax.ShapeDtypeStruct((n, h2 // 2), jnp.float32)] * 2,
    )


@functools.lru_cache(maxsize=None)
def _tc_mid(n, c, h_out, bm):
    def body(p0, p1, g0, g1, d0, d1, bb, w, o):
        dis = _dis(d0, d1)
        h = jnp.concatenate([p0[...] + g0[...], p1[...] + g1[...]], axis=1) * dis
        h = h + bb[...]
        o[...] = jnp.dot(h, w[...], preferred_element_type=jnp.float32) * dis

    return pl.pallas_call(
        body,
        grid=(n // bm,),
        in_specs=[
            pl.BlockSpec((bm, c), lambda m: (m, 0)),
            pl.BlockSpec((bm, c), lambda m: (m, 0)),
            pl.BlockSpec((bm, c), lambda m: (m, 0)),
            pl.BlockSpec((bm, c), lambda m: (m, 0)),
            pl.BlockSpec((bm, 128), lambda m: (m, 0)),
            pl.BlockSpec((bm, 128), lambda m: (m, 0)),
            pl.BlockSpec((1, 2 * c), lambda m: (0, 0)),
            pl.BlockSpec((2 * c, h_out), lambda m: (0, 0)),
        ],
        out_specs=pl.BlockSpec((bm, h_out), lambda m: (m, 0)),
        out_shape=jax.ShapeDtypeStruct((n, h_out), jnp.float32),
    )


@functools.lru_cache(maxsize=None)
def _tc_hs3(n, c, bm):
    def body(p0, p1, g, d0, d1, bb, o):
        dis = _dis(d0, d1)
        h = (p0[...] + p1[...] + g[...]) * dis
        o[...] = (h + bb[...]) * dis

    return pl.pallas_call(
        body,
        grid=(n // bm,),
        in_specs=[
            pl.BlockSpec((bm, c), lambda m: (m, 0)),
            pl.BlockSpec((bm, c), lambda m: (m, 0)),
            pl.BlockSpec((bm, c), lambda m: (m, 0)),
            pl.BlockSpec((bm, 128), lambda m: (m, 0)),
            pl.BlockSpec((bm, 128), lambda m: (m, 0)),
            pl.BlockSpec((1, c), lambda m: (0, 0)),
        ],
        out_specs=pl.BlockSpec((bm, c), lambda m: (m, 0)),
        out_shape=jax.ShapeDtypeStruct((n, c), jnp.float32),
    )


@functools.lru_cache(maxsize=None)
def _tc_head(n, c, h4, bm, max_logstd):
    def body(p0, p1, g, d0, d1, wmu, bmu, wlv, blv, nz, pa,
             z_ref, mu_ref, lv_ref):
        dis = _dis(d0, d1)
        sh = (p0[...] + p1[...] + g[...]) * dis
        mu = jnp.dot(sh, wmu[...], preferred_element_type=jnp.float32) + bmu[...]
        logvar = jnp.dot(sh, wlv[...], preferred_element_type=jnp.float32) + blv[...]
        lvc = jnp.minimum(logvar, max_logstd)
        z = mu + nz[...] * jnp.exp(0.5 * lvc)
        z = jnp.where(z >= 0, z, pa[...] * z)
        z_ref[...] = z
        mu_ref[...] = mu
        lv_ref[...] = logvar

    return pl.pallas_call(
        body,
        grid=(n // bm,),
        in_specs=[
            pl.BlockSpec((bm, c), lambda m: (m, 0)),
            pl.BlockSpec((bm, c), lambda m: (m, 0)),
            pl.BlockSpec((bm, c), lambda m: (m, 0)),
            pl.BlockSpec((bm, 128), lambda m: (m, 0)),
            pl.BlockSpec((bm, 128), lambda m: (m, 0)),
            pl.BlockSpec((c, h4), lambda m: (0, 0)),
            pl.BlockSpec((1, h4), lambda m: (0, 0)),
            pl.BlockSpec((c, h4), lambda m: (0, 0)),
            pl.BlockSpec((1, h4), lambda m: (0, 0)),
            pl.BlockSpec((bm, h4), lambda m: (m, 0)),
            pl.BlockSpec((1, h4), lambda m: (0, 0)),
        ],
        out_specs=[pl.BlockSpec((bm, h4), lambda m: (m, 0))] * 3,
        out_shape=[jax.ShapeDtypeStruct((n, h4), jnp.float32)] * 3,
    )


def kernel(x, edge_index, edge_attr, W1, b1, W2, b2, W3, b3,
           Wmu, bmu, Wlv, blv, prelu_a, noise):
    n, in_ch = x.shape
    e = edge_index.shape[1]
    h1, h2 = W1.shape[1], W2.shape[1]
    h3, h4 = W3.shape[1], Wmu.shape[1]
    bm = 1000

    ei = edge_index.astype(jnp.int32)
    row = ei[0]
    col = ei[1]
    w = edge_attr.astype(jnp.float32)

    d0, d1 = _sc_deg(n, e)(col, w)
    xs0, xs1 = _tc_pre(n, in_ch, bm)(x, d0, d1)
    p10, p11 = _sc_agg_csplit(n, e)(xs0, xs1, row, col, w)
    g20, g21 = _tc_l12(n, in_ch // 2, h1, h2, bm)(
        p10, p11, xs0, xs1, d0, d1, W1, b1.reshape(1, -1), W2)
    p20, p21 = _sc_agg_csplit(n, e)(g20, g21, row, col, w)
    gs3 = _tc_mid(n, h2 // 2, h3, bm)(
        p20, p21, g20, g21, d0, d1, b2.reshape(1, -1), W3)
    p30, p31 = _sc_agg_esplit(n, e)(gs3, row, col, w)
    hs3 = _tc_hs3(n, h3, bm)(p30, p31, gs3, d0, d1, b3.reshape(1, -1))
    p40, p41 = _sc_agg_esplit(n, e)(hs3, row, col, w)
    z, mu, logvar = _tc_head(n, h3, h4, bm, 20.0)(
        p40, p41, hs3, d0, d1, Wmu, bmu.reshape(1, -1),
        Wlv, blv.reshape(1, -1), noise, prelu_a.reshape(1, -1))
    return z, mu, logvar


# speedup vs baseline: 1.0563x; 1.0563x over previous
import functools

import jax
import jax.numpy as jnp
from jax import lax
from jax.experimental import pallas as pl
from jax.experimental.pallas import tpu as pltpu
from jax.experimental.pallas import tpu_sc as plsc

_NSUB = 16


def _row_split(n):
    ra = ((n // _NSUB) + 7) & ~7
    rb = n - (_NSUB - 1) * ra
    assert rb > 0 and rb % 8 == 0
    return ra, rb


def _zero_rows(zb, acc, sid, n, zr, sem):
    ra, rb = _row_split(n)

    def mk(base, cnt):
        def zissue(k, carry):
            pltpu.async_copy(zb, acc.at[pl.ds(base + k * zr, zr)], sem)
            return carry
        lax.fori_loop(0, cnt // zr, zissue, 0)

        def zwait(k, carry):
            pltpu.make_async_copy(zb, acc.at[pl.ds(0, zr)], sem).wait()
            return carry
        lax.fori_loop(0, cnt // zr, zwait, 0)

    @pl.when(sid < _NSUB - 1)
    def _():
        mk(sid * ra, ra)

    @pl.when(sid == _NSUB - 1)
    def _():
        mk((_NSUB - 1) * ra, rb)


def _drain_rows(acc, p, sid, n):
    ra, rb = _row_split(n)

    @pl.when(sid < _NSUB - 1)
    def _():
        pltpu.sync_copy(acc.at[pl.ds(sid * ra, ra)], p.at[pl.ds(sid * ra, ra)])

    @pl.when(sid == _NSUB - 1)
    def _():
        pltpu.sync_copy(acc.at[pl.ds((_NSUB - 1) * ra, rb)],
                        p.at[pl.ds((_NSUB - 1) * ra, rb)])


def _lane_bcast(w16, lane):
    return w16.at[jnp.full((16,), lane, jnp.int32)].get(
        mode="promise_in_bounds")


def _groups(B):
    out, s = [], 0
    while s + 16 <= B:
        out.append((s, 0))
        s += 16
    if s < B:
        out.append((B - 16, 16 - (B - s)))
    return out


def _zero_zb(zb, zr):
    zero16 = jnp.zeros((16,), jnp.float32)
    for r in range(zr):
        for j in range(8):
            zb.at[r][pl.ds(j * 16, 16)] = zero16


_NS = 3


def _edge_pipeline(y, rows_v, col_hbm, w_hbm, acc, colbs, wbs, gbs,
                   sgs, sws, scs, sss, e0, ept, B, gather):
    nb = ept // B
    assert nb >= _NS + 1

    def issue(t, s):
        pltpu.async_copy(col_hbm.at[pl.ds(e0 + t * B, B)], colbs[s], scs[s])
        pltpu.async_copy(w_hbm.at[pl.ds(e0 + t * B, B)], wbs[s], sws[s])
        if gather:
            pltpu.async_copy(y.at[rows_v.at[pl.ds(t * B, B)]], gbs[s], sgs[s])

    def wait_gather(s):
        pltpu.make_async_copy(y.at[pl.ds(0, B)], gbs[s], sgs[s]).wait()

    def wait_col(s):
        pltpu.make_async_copy(col_hbm.at[pl.ds(0, B)], colbs[s], scs[s]).wait()

    def wait_w(s):
        pltpu.make_async_copy(w_hbm.at[pl.ds(0, B)], wbs[s], sws[s]).wait()

    def wait_scat(s):
        pltpu.make_async_copy(gbs[s], acc.at[pl.ds(0, B)], sss[s]).wait()

    def batch(u, s, in_loop):
        if gather:
            wait_gather(s)
        wait_w(s)
        for start, lo in _groups(B):
            w16 = wbs[s][pl.ds(start, 16)]
            for lane in range(lo, 16):
                wv = _lane_bcast(w16, lane)
                r = gbs[s].at[start + lane]
                if gather:
                    for j in range(8):
                        r[pl.ds(j * 16, 16)] = r[pl.ds(j * 16, 16)] * wv
                else:
                    r[pl.ds(0, 16)] = wv
        wait_col(s)
        pltpu.async_copy(gbs[s], acc.at[colbs[s]], sss[s], add=True)
        v = u + 2
        s2 = (s + 2) % _NS
        if in_loop:
            @pl.when(v < nb)
            def _():
                @pl.when(v >= _NS)
                def _():
                    wait_scat(s2)
                issue(v, s2)
        else:
            if v < nb:
                if v >= _NS:
                    wait_scat(s2)
                issue(v, s2)

    for t in range(2):
        issue(t, t)

    def tri(q, carry):
        for s in range(_NS):
            batch(q * _NS + s, s, True)
        return carry
    nq = nb // _NS
    lax.fori_loop(0, nq, tri, 0)
    for k in range(nb % _NS):
        batch(nq * _NS + k, k, False)
    for s in range(_NS):
        wait_scat(s)


@functools.lru_cache(maxsize=None)
def _sc_deg(n, e):
    ncores = 2
    ept = e // (_NSUB * ncores)
    B = 40
    nb = ept // B
    zr = 8
    mesh = plsc.VectorSubcoreMesh(core_axis_name="c", subcore_axis_name="s")

    def body(col_hbm, w_hbm, p0, p1, *scr):
        colbs, wbs, gbs = scr[0:3], scr[3:6], scr[6:9]
        zb, acc = scr[9], scr[10]
        sws, scs, sss = scr[11:14], scr[14:17], scr[17:20]
        cid = lax.axis_index("c")
        sid = lax.axis_index("s")
        _zero_zb(zb, zr)
        _zero_rows(zb, acc, sid, n, zr, sss[0])
        plsc.subcore_barrier()

        wid = cid * _NSUB + sid
        e0 = wid * ept
        _edge_pipeline(None, None, col_hbm, w_hbm, acc,
                       colbs, wbs, gbs,
                       None, sws, scs, sss, e0, ept, B, gather=False)
        plsc.subcore_barrier()

        @pl.when(cid == 0)
        def _():
            _drain_rows(acc, p0, sid, n)

        @pl.when(cid == 1)
        def _():
            _drain_rows(acc, p1, sid, n)

    return pl.kernel(
        body,
        mesh=mesh,
        out_type=[jax.ShapeDtypeStruct((n, 128), jnp.float32)] * 2,
        scratch_types=(
            [pltpu.VMEM((B,), jnp.int32)] * 3
            + [pltpu.VMEM((B,), jnp.float32)] * 3
            + [pltpu.VMEM((B, 128), jnp.float32)] * 3
            + [pltpu.VMEM((zr, 128), jnp.float32),
               pltpu.VMEM_SHARED((n, 128), jnp.float32)]
            + [pltpu.SemaphoreType.DMA] * 9
        ),
    )


@functools.lru_cache(maxsize=None)
def _sc_agg_csplit(n, e):
    ept = e // _NSUB
    B = 80
    zr = 8
    mesh = plsc.VectorSubcoreMesh(core_axis_name="c", subcore_axis_name="s")

    def body(y0, y1, row_hbm, col_hbm, w_hbm, p0, p1, *scr):
        rows_v = scr[0]
        colbs, wbs, gbs = scr[1:4], scr[4:7], scr[7:10]
        zb, acc = scr[10], scr[11]
        sgs, sws, scs, sss = (scr[12:15], scr[15:18], scr[18:21],
                              scr[21:24])
        cid = lax.axis_index("c")
        sid = lax.axis_index("s")
        _zero_zb(zb, zr)
        _zero_rows(zb, acc, sid, n, zr, sss[0])

        e0 = sid * ept
        pltpu.sync_copy(row_hbm.at[pl.ds(e0, ept)], rows_v)
        plsc.subcore_barrier()

        def run(y):
            _edge_pipeline(y, rows_v, col_hbm, w_hbm, acc,
                           colbs, wbs, gbs,
                           sgs, sws, scs, sss, e0, ept, B, gather=True)

        @pl.when(cid == 0)
        def _():
            run(y0)

        @pl.when(cid == 1)
        def _():
            run(y1)
        plsc.subcore_barrier()

        @pl.when(cid == 0)
        def _():
            _drain_rows(acc, p0, sid, n)

        @pl.when(cid == 1)
        def _():
            _drain_rows(acc, p1, sid, n)

    return pl.kernel(
        body,
        mesh=mesh,
        out_type=[jax.ShapeDtypeStruct((n, 128), jnp.float32)] * 2,
        scratch_types=(
            [pltpu.VMEM((ept,), jnp.int32)]
            + [pltpu.VMEM((B,), jnp.int32)] * 3
            + [pltpu.VMEM((B,), jnp.float32)] * 3
            + [pltpu.VMEM((B, 128), jnp.float32)] * 3
            + [pltpu.VMEM((zr, 128), jnp.float32),
               pltpu.VMEM_SHARED((n, 128), jnp.float32)]
            + [pltpu.SemaphoreType.DMA] * 12
        ),
    )


@functools.lru_cache(maxsize=None)
def _sc_agg_esplit(n, e):
    ncores = 2
    ept = e // (_NSUB * ncores)
    B = 40
    zr = 8
    mesh = plsc.VectorSubcoreMesh(core_axis_name="c", subcore_axis_name="s")

    def body(y, row_hbm, col_hbm, w_hbm, p0, p1, *scr):
        rows_v = scr[0]
        colbs, wbs, gbs = scr[1:4], scr[4:7], scr[7:10]
        zb, acc = scr[10], scr[11]
        sgs, sws, scs, sss = (scr[12:15], scr[15:18], scr[18:21],
                              scr[21:24])
        cid = lax.axis_index("c")
        sid = lax.axis_index("s")
        _zero_zb(zb, zr)
        _zero_rows(zb, acc, sid, n, zr, sss[0])

        wid = cid * _NSUB + sid
        e0 = wid * ept
        pltpu.sync_copy(row_hbm.at[pl.ds(e0, ept)], rows_v)
        plsc.subcore_barrier()

        _edge_pipeline(y, rows_v, col_hbm, w_hbm, acc,
                       colbs, wbs, gbs,
                       sgs, sws, scs, sss, e0, ept, B, gather=True)
        plsc.subcore_barrier()

        @pl.when(cid == 0)
        def _():
            _drain_rows(acc, p0, sid, n)

        @pl.when(cid == 1)
        def _():
            _drain_rows(acc, p1, sid, n)

    return pl.kernel(
        body,
        mesh=mesh,
        out_type=[jax.ShapeDtypeStruct((n, 128), jnp.float32)] * 2,
        scratch_types=(
            [pltpu.VMEM((ept,), jnp.int32)]
            + [pltpu.VMEM((B,), jnp.int32)] * 3
            + [pltpu.VMEM((B,), jnp.float32)] * 3
            + [pltpu.VMEM((B, 128), jnp.float32)] * 3
            + [pltpu.VMEM((zr, 128), jnp.float32),
               pltpu.VMEM_SHARED((n, 128), jnp.float32)]
            + [pltpu.SemaphoreType.DMA] * 12
        ),
    )


def _dis(d0, d1):
    return lax.rsqrt(d0[:, 0:1] + d1[:, 0:1] + 1.0)


@functools.lru_cache(maxsize=None)
def _tc_pre(n, in_ch, bm):
    half = in_ch // 2

    def body(x_ref, d0, d1, xs0, xs1):
        xs = x_ref[...] * _dis(d0, d1)
        xs0[...] = xs[:, :half]
        xs1[...] = xs[:, half:]

    return pl.pallas_call(
        body,
        grid=(n // bm,),
        in_specs=[
            pl.BlockSpec((bm, in_ch), lambda m: (m, 0)),
            pl.BlockSpec((bm, 128), lambda m: (m, 0)),
            pl.BlockSpec((bm, 128), lambda m: (m, 0)),
        ],
        out_specs=[pl.BlockSpec((bm, half), lambda m: (m, 0))] * 2,
        out_shape=[jax.ShapeDtypeStruct((n, half), jnp.float32)] * 2,
    )


@functools.lru_cache(maxsize=None)
def _tc_l12(n, c, h1, h2, bm):
    def body(p0, p1, x0, x1, d0, d1, w1, bb1, w2, o0, o1):
        dis = _dis(d0, d1)
        a = jnp.concatenate([p0[...] + x0[...], p1[...] + x1[...]], axis=1) * dis
        hh = jnp.dot(a, w1[...], preferred_element_type=jnp.float32) + bb1[...]
        g = jnp.dot(hh, w2[...], preferred_element_type=jnp.float32) * dis
        o0[...] = g[:, : h2 // 2]
        o1[...] = g[:, h2 // 2:]

    return pl.pallas_call(
        body,
        grid=(n // bm,),
        in_specs=[
            pl.BlockSpec((bm, c), lambda m: (m, 0)),
            pl.BlockSpec((bm, c), lambda m: (m, 0)),
            pl.BlockSpec((bm, c), lambda m: (m, 0)),
            pl.BlockSpec((bm, c), lambda m: (m, 0)),
            pl.BlockSpec((bm, 128), lambda m: (m, 0)),
            pl.BlockSpec((bm, 128), lambda m: (m, 0)),
            pl.BlockSpec((2 * c, h1), lambda m: (0, 0)),
            pl.BlockSpec((1, h1), lambda m: (0, 0)),
            pl.BlockSpec((h1, h2), lambda m: (0, 0)),
        ],
        out_specs=[pl.BlockSpec((bm, h2 // 2), lambda m: (m, 0))] * 2,
        out_shape=[jax.ShapeDtypeStruct((n, h2 // 2), jnp.float32)] * 2,
    )


@functools.lru_cache(maxsize=None)
def _tc_mid(n, c, h_out, bm):
    def body(p0, p1, g0, g1, d0, d1, bb, w, o):
        dis = _dis(d0, d1)
        h = jnp.concatenate([p0[...] + g0[...], p1[...] + g1[...]], axis=1) * dis
        h = h + bb[...]
        o[...] = jnp.dot(h, w[...], preferred_element_type=jnp.float32) * dis

    return pl.pallas_call(
        body,
        grid=(n // bm,),
        in_specs=[
            pl.BlockSpec((bm, c), lambda m: (m, 0)),
            pl.BlockSpec((bm, c), lambda m: (m, 0)),
            pl.BlockSpec((bm, c), lambda m: (m, 0)),
            pl.BlockSpec((bm, c), lambda m: (m, 0)),
            pl.BlockSpec((bm, 128), lambda m: (m, 0)),
            pl.BlockSpec((bm, 128), lambda m: (m, 0)),
            pl.BlockSpec((1, 2 * c), lambda m: (0, 0)),
            pl.BlockSpec((2 * c, h_out), lambda m: (0, 0)),
        ],
        out_specs=pl.BlockSpec((bm, h_out), lambda m: (m, 0)),
        out_shape=jax.ShapeDtypeStruct((n, h_out), jnp.float32),
    )


@functools.lru_cache(maxsize=None)
def _tc_hs3(n, c, bm):
    def body(p0, p1, g, d0, d1, bb, o):
        dis = _dis(d0, d1)
        h = (p0[...] + p1[...] + g[...]) * dis
        o[...] = (h + bb[...]) * dis

    return pl.pallas_call(
        body,
        grid=(n // bm,),
        in_specs=[
            pl.BlockSpec((bm, c), lambda m: (m, 0)),
            pl.BlockSpec((bm, c), lambda m: (m, 0)),
            pl.BlockSpec((bm, c), lambda m: (m, 0)),
            pl.BlockSpec((bm, 128), lambda m: (m, 0)),
            pl.BlockSpec((bm, 128), lambda m: (m, 0)),
            pl.BlockSpec((1, c), lambda m: (0, 0)),
        ],
        out_specs=pl.BlockSpec((bm, c), lambda m: (m, 0)),
        out_shape=jax.ShapeDtypeStruct((n, c), jnp.float32),
    )


@functools.lru_cache(maxsize=None)
def _tc_head(n, c, h4, bm, max_logstd):
    def body(p0, p1, g, d0, d1, wmu, bmu, wlv, blv, nz, pa,
             z_ref, mu_ref, lv_ref):
        dis = _dis(d0, d1)
        sh = (p0[...] + p1[...] + g[...]) * dis
        mu = jnp.dot(sh, wmu[...], preferred_element_type=jnp.float32) + bmu[...]
        logvar = jnp.dot(sh, wlv[...], preferred_element_type=jnp.float32) + blv[...]
        lvc = jnp.minimum(logvar, max_logstd)
        z = mu + nz[...] * jnp.exp(0.5 * lvc)
        z = jnp.where(z >= 0, z, pa[...] * z)
        z_ref[...] = z
        mu_ref[...] = mu
        lv_ref[...] = logvar

    return pl.pallas_call(
        body,
        grid=(n // bm,),
        in_specs=[
            pl.BlockSpec((bm, c), lambda m: (m, 0)),
            pl.BlockSpec((bm, c), lambda m: (m, 0)),
            pl.BlockSpec((bm, c), lambda m: (m, 0)),
            pl.BlockSpec((bm, 128), lambda m: (m, 0)),
            pl.BlockSpec((bm, 128), lambda m: (m, 0)),
            pl.BlockSpec((c, h4), lambda m: (0, 0)),
            pl.BlockSpec((1, h4), lambda m: (0, 0)),
            pl.BlockSpec((c, h4), lambda m: (0, 0)),
            pl.BlockSpec((1, h4), lambda m: (0, 0)),
            pl.BlockSpec((bm, h4), lambda m: (m, 0)),
            pl.BlockSpec((1, h4), lambda m: (0, 0)),
        ],
        out_specs=[pl.BlockSpec((bm, h4), lambda m: (m, 0))] * 3,
        out_shape=[jax.ShapeDtypeStruct((n, h4), jnp.float32)] * 3,
    )


def kernel(x, edge_index, edge_attr, W1, b1, W2, b2, W3, b3,
           Wmu, bmu, Wlv, blv, prelu_a, noise):
    n, in_ch = x.shape
    e = edge_index.shape[1]
    h1, h2 = W1.shape[1], W2.shape[1]
    h3, h4 = W3.shape[1], Wmu.shape[1]
    bm = 1000

    ei = edge_index.astype(jnp.int32)
    row = ei[0]
    col = ei[1]
    w = edge_attr.astype(jnp.float32)

    d0, d1 = _sc_deg(n, e)(col, w)
    xs0, xs1 = _tc_pre(n, in_ch, bm)(x, d0, d1)
    p10, p11 = _sc_agg_csplit(n, e)(xs0, xs1, row, col, w)
    g20, g21 = _tc_l12(n, in_ch // 2, h1, h2, bm)(
        p10, p11, xs0, xs1, d0, d1, W1, b1.reshape(1, -1), W2)
    p20, p21 = _sc_agg_csplit(n, e)(g20, g21, row, col, w)
    gs3 = _tc_mid(n, h2 // 2, h3, bm)(
        p20, p21, g20, g21, d0, d1, b2.reshape(1, -1), W3)
    p30, p31 = _sc_agg_esplit(n, e)(gs3, row, col, w)
    hs3 = _tc_hs3(n, h3, bm)(p30, p31, gs3, d0, d1, b3.reshape(1, -1))
    p40, p41 = _sc_agg_esplit(n, e)(hs3, row, col, w)
    z, mu, logvar = _tc_head(n, h3, h4, bm, 20.0)(
        p40, p41, hs3, d0, d1, Wmu, bmu.reshape(1, -1),
        Wlv, blv.reshape(1, -1), noise, prelu_a.reshape(1, -1))
    return z, mu, logvar
